# Initial kernel scaffold; baseline (speedup 1.0000x reference)
import jax
import jax.numpy as jnp
from jax.experimental import pallas as pl


def kernel(position, z, transition, dir_locs, dir_covs, t, displacement, W1, b1, W2, b2):
    raise NotImplementedError("write your pallas kernel here")


# speedup vs baseline: 2.4134x; 2.4134x over previous
import jax
import jax.numpy as jnp
import numpy as np
from jax import lax
from jax.experimental import pallas as pl
from jax.experimental.pallas import tpu as pltpu

P = 131072
M = P // 4
MB = 2048
GRID = M // MB

K1 = (np.uint32(1832780943), np.uint32(270669613))
K2 = (np.uint32(64467757), np.uint32(2916123636))
K3 = (np.uint32(2465931498), np.uint32(255383827))

_TINY = np.float32(np.finfo(np.float32).tiny)
_LO = np.nextafter(np.float32(-1.0), np.float32(0.0))
_SPAN_N = np.float32(1.0) - _LO
_SQRT2 = np.float32(np.sqrt(2.0))

_ROTS = (13, 15, 26, 6, 17, 29, 16, 24, 13, 15, 26, 6, 17, 29, 16, 24,
         13, 15, 26, 6)


def _threefry_bits(ka, kb, cnt, kconst):
    if kconst is not None:
        kc = np.uint32(kconst[0] ^ kconst[1] ^ np.uint32(0x1BD11BDA))
    else:
        kc = ka ^ kb ^ np.uint32(0x1BD11BDA)
    ks = (ka, kb, kc)
    x0 = jnp.broadcast_to(ka, cnt.shape).astype(jnp.uint32)
    x1 = cnt + kb
    for i in range(5):
        for r in _ROTS[4 * i:4 * i + 4]:
            x0 = x0 + x1
            x1 = (x1 << np.uint32(r)) | (x1 >> np.uint32(32 - r))
            x1 = x0 ^ x1
        x0 = x0 + ks[(i + 1) % 3]
        x1 = x1 + ks[(i + 2) % 3] + np.uint32(i + 1)
    return x0 ^ x1


def _unit_float(bits):
    fb = (bits >> np.uint32(9)) | np.uint32(0x3F800000)
    return lax.bitcast_convert_type(fb, jnp.float32) - np.float32(1.0)


def _argmax4(s0, s1, s2, s3):
    m = jnp.maximum(jnp.maximum(s0, s1), jnp.maximum(s2, s3))
    return jnp.where(s0 == m, np.int32(0),
                     jnp.where(s1 == m, np.int32(1),
                               jnp.where(s2 == m, np.int32(2), np.int32(3))))


def _sel4(z, r0, r1, r2, r3):
    return jnp.where(z == 0, r0, jnp.where(z == 1, r1, jnp.where(z == 2, r2, r3)))


def _body(disp, w1, b1, w2, b2, tt, dl, dc, pos, np_ref, z_ref):
    i = pl.program_id(0)
    base = (i * MB).astype(jnp.uint32)

    bf = lambda v: v.astype(jnp.bfloat16).astype(jnp.float32)
    db = [bf(disp[0]), bf(disp[1])]
    h = [(db[0] * bf(w1[k, 0]) + db[1] * bf(w1[k, 1])) + b1[k] for k in range(4)]
    h = [v / (np.float32(1.0) + jnp.abs(v)) for v in h]
    hb = [bf(v) for v in h]
    raw = [(((hb[0] * bf(w2[j, 0]) + hb[1] * bf(w2[j, 1])) + hb[2] * bf(w2[j, 2]))
            + hb[3] * bf(w2[j, 3])) + b2[j] for j in range(4)]
    amax = jnp.maximum(jnp.maximum(raw[0], raw[1]),
                       jnp.maximum(raw[2], raw[3]))
    ex = [jnp.exp(v - amax) for v in raw]
    lse = jnp.log(((ex[0] + ex[1]) + ex[2]) + ex[3]) + amax
    pre = [v - lse for v in raw]

    rit = lax.broadcasted_iota(jnp.int32, (32, MB), 0).astype(jnp.uint32)
    lit = lax.broadcasted_iota(jnp.int32, (32, MB), 1).astype(jnp.uint32)
    cnt = np.uint32(16) * (base + lit) + (rit & np.uint32(15))
    row_lt16 = rit < np.uint32(16)
    ka = jnp.where(row_lt16, K1[0], K2[0]).astype(jnp.uint32)
    kb = jnp.where(row_lt16, K1[1], K2[1]).astype(jnp.uint32)
    u = _unit_float(_threefry_bits(ka, kb, cnt, None))
    u = jnp.maximum(_TINY, u * (np.float32(1.0) - _TINY) + _TINY)
    e = -jnp.log(u)

    le = jnp.log(e[0:16])
    le2 = jnp.log(e[16:32])

    rit8 = lax.broadcasted_iota(jnp.int32, (8, MB), 0).astype(jnp.uint32)
    lit8 = lax.broadcasted_iota(jnp.int32, (8, MB), 1).astype(jnp.uint32)
    cnt8 = np.uint32(8) * (base + lit8) + rit8
    un = _unit_float(_threefry_bits(K3[0], K3[1], cnt8, K3))
    un = jnp.maximum(_LO, un * _SPAN_N + _LO)
    eps = _SQRT2 * lax.erf_inv(un)

    tta = tt[...]
    dla = dl[...]
    dca = dc[...]
    posa = pos[...]
    z_rows = []
    np_rows = [None] * 8

    for q in range(4):
        s = [pre[j] - le[4 * q + j:4 * q + j + 1] for j in range(4)]
        zp = _argmax4(*s)

        tsel = [_sel4(zp, tta[16 * q + j:16 * q + j + 1],
                      tta[16 * q + 4 + j:16 * q + 5 + j],
                      tta[16 * q + 8 + j:16 * q + 9 + j],
                      tta[16 * q + 12 + j:16 * q + 13 + j]) for j in range(4)]
        s2 = [jnp.log(tsel[j]) - le2[4 * q + j:4 * q + j + 1] for j in range(4)]
        zc = _argmax4(*s2)
        z_rows.append(zc)

        loc0 = _sel4(zc, *[dla[8 * q + 2 * k:8 * q + 2 * k + 1] for k in range(4)])
        loc1 = _sel4(zc, *[dla[8 * q + 2 * k + 1:8 * q + 2 * k + 2] for k in range(4)])
        c00 = _sel4(zc, *[dca[16 * q + 4 * k:16 * q + 4 * k + 1] for k in range(4)])
        c10 = _sel4(zc, *[dca[16 * q + 4 * k + 2:16 * q + 4 * k + 3] for k in range(4)])
        c11 = _sel4(zc, *[dca[16 * q + 4 * k + 3:16 * q + 4 * k + 4] for k in range(4)])

        e00 = jnp.exp(c00)
        e11 = jnp.exp(c11)
        ep0 = eps[2 * q:2 * q + 1]
        ep1 = eps[2 * q + 1:2 * q + 2]
        v0 = loc0 + e00 * ep0
        v1 = loc1 + (c10 * ep0 + e11 * ep1)
        np_rows[2 * q] = posa[2 * q:2 * q + 1] + v0
        np_rows[2 * q + 1] = posa[2 * q + 1:2 * q + 2] + v1

    np_ref[...] = jnp.concatenate(np_rows, axis=0)
    z_ref[...] = jnp.concatenate(z_rows, axis=0)


@jax.jit
def kernel(position, z, transition, dir_locs, dir_covs, t, displacement,
           W1, b1, W2, b2):
    del z, t
    tt = transition.reshape(M, 4, 4, 4).transpose(1, 2, 3, 0).reshape(64, M)
    dl = dir_locs.reshape(M, 4, 4, 2).transpose(1, 2, 3, 0).reshape(32, M)
    dc = dir_covs.reshape(M, 4, 4, 2, 2).transpose(1, 2, 3, 4, 0).reshape(64, M)
    pos = position.reshape(M, 4, 2).transpose(1, 2, 0).reshape(8, M)

    smem = pl.BlockSpec(memory_space=pltpu.SMEM)
    col = lambda rows: pl.BlockSpec((rows, MB), lambda i: (0, i))
    new_pos_t, z_t = pl.pallas_call(
        _body,
        grid=(GRID,),
        in_specs=[smem, smem, smem, smem, smem,
                  col(64), col(32), col(64), col(8)],
        out_specs=[col(8), col(4)],
        out_shape=[jax.ShapeDtypeStruct((8, M), jnp.float32),
                   jax.ShapeDtypeStruct((4, M), jnp.int32)],
        compiler_params=pltpu.CompilerParams(
            dimension_semantics=("parallel",)),
    )(displacement, W1, b1, W2, b2, tt, dl, dc, pos)

    new_position = new_pos_t.reshape(4, 2, M).transpose(2, 0, 1).reshape(P, 2)
    z_current = z_t.transpose(1, 0).reshape(P)
    return (new_position, z_current, transition, dir_locs, dir_covs)


# speedup vs baseline: 10.1425x; 4.2026x over previous
import jax
import jax.numpy as jnp
import numpy as np
from jax import lax
from jax.experimental import pallas as pl
from jax.experimental.pallas import tpu as pltpu

P = 131072
N = 8192
GRID = P // N

K1 = (np.uint32(1832780943), np.uint32(270669613))
K2 = (np.uint32(64467757), np.uint32(2916123636))
K3 = (np.uint32(2465931498), np.uint32(255383827))

_TINY = np.float32(np.finfo(np.float32).tiny)
_LO = np.nextafter(np.float32(-1.0), np.float32(0.0))
_SPAN_N = np.float32(1.0) - _LO
_SQRT2 = np.float32(np.sqrt(2.0))

_ROTS = (13, 15, 26, 6, 17, 29, 16, 24, 13, 15, 26, 6, 17, 29, 16, 24,
         13, 15, 26, 6)


def _threefry_bits(ka, kb, cnt, kconst):
    if kconst is not None:
        kc = np.uint32(kconst[0] ^ kconst[1] ^ np.uint32(0x1BD11BDA))
    else:
        kc = ka ^ kb ^ np.uint32(0x1BD11BDA)
    ks = (ka, kb, kc)
    x0 = jnp.broadcast_to(ka, cnt.shape).astype(jnp.uint32)
    x1 = cnt + kb
    for i in range(5):
        for r in _ROTS[4 * i:4 * i + 4]:
            x0 = x0 + x1
            x1 = (x1 << np.uint32(r)) | (x1 >> np.uint32(32 - r))
            x1 = x0 ^ x1
        x0 = x0 + ks[(i + 1) % 3]
        x1 = x1 + ks[(i + 2) % 3] + np.uint32(i + 1)
    return x0 ^ x1


def _unit_float(bits):
    fb = (bits >> np.uint32(9)) | np.uint32(0x3F800000)
    return lax.bitcast_convert_type(fb, jnp.float32) - np.float32(1.0)


def _argmax4(s0, s1, s2, s3):
    m = jnp.maximum(jnp.maximum(s0, s1), jnp.maximum(s2, s3))
    return jnp.where(s0 == m, np.int32(0),
                     jnp.where(s1 == m, np.int32(1),
                               jnp.where(s2 == m, np.int32(2), np.int32(3))))


def _sel4(z, r0, r1, r2, r3):
    return jnp.where(z == 0, r0, jnp.where(z == 1, r1, jnp.where(z == 2, r2, r3)))


def _body(disp, w1, b1, w2, b2, tr, dl, dc, pos, np_ref, z_ref):
    i = pl.program_id(0)
    base = (i * N).astype(jnp.uint32)

    bf = lambda v: v.astype(jnp.bfloat16).astype(jnp.float32)
    db = [bf(disp[0]), bf(disp[1])]
    h = [(db[0] * bf(w1[k, 0]) + db[1] * bf(w1[k, 1])) + b1[k] for k in range(4)]
    h = [v / (np.float32(1.0) + jnp.abs(v)) for v in h]
    hb = [bf(v) for v in h]
    raw = [(((hb[0] * bf(w2[j, 0]) + hb[1] * bf(w2[j, 1])) + hb[2] * bf(w2[j, 2]))
            + hb[3] * bf(w2[j, 3])) + b2[j] for j in range(4)]
    amax = jnp.maximum(jnp.maximum(raw[0], raw[1]),
                       jnp.maximum(raw[2], raw[3]))
    ex = [jnp.exp(v - amax) for v in raw]
    lse = jnp.log(((ex[0] + ex[1]) + ex[2]) + ex[3]) + amax
    pre = [v - lse for v in raw]

    rit = lax.broadcasted_iota(jnp.int32, (8, N), 0).astype(jnp.uint32)
    lit = lax.broadcasted_iota(jnp.int32, (8, N), 1).astype(jnp.uint32)
    cnt = np.uint32(4) * (base + lit) + (rit & np.uint32(3))
    row_lt4 = rit < np.uint32(4)
    ka = jnp.where(row_lt4, K1[0], K2[0]).astype(jnp.uint32)
    kb = jnp.where(row_lt4, K1[1], K2[1]).astype(jnp.uint32)
    u = _unit_float(_threefry_bits(ka, kb, cnt, None))
    u = jnp.maximum(_TINY, u * (np.float32(1.0) - _TINY) + _TINY)
    e = -jnp.log(u)
    le = jnp.log(e[0:4])
    le2 = jnp.log(e[4:8])

    rit2 = lax.broadcasted_iota(jnp.int32, (2, N), 0).astype(jnp.uint32)
    lit2 = lax.broadcasted_iota(jnp.int32, (2, N), 1).astype(jnp.uint32)
    cnt2 = np.uint32(2) * (base + lit2) + rit2
    un = _unit_float(_threefry_bits(K3[0], K3[1], cnt2, K3))
    un = jnp.maximum(_LO, un * _SPAN_N + _LO)
    eps = _SQRT2 * lax.erf_inv(un)

    tta = jnp.transpose(tr[...])
    dla = jnp.transpose(dl[...])
    dca = jnp.transpose(dc[...])
    posa = jnp.transpose(pos[...])

    s = [pre[j] - le[j:j + 1] for j in range(4)]
    zp = _argmax4(*s)
    tsel = [_sel4(zp, tta[j:j + 1], tta[4 + j:5 + j],
                  tta[8 + j:9 + j], tta[12 + j:13 + j]) for j in range(4)]
    s2 = [jnp.log(tsel[j]) - le2[j:j + 1] for j in range(4)]
    zc = _argmax4(*s2)

    loc0 = _sel4(zc, *[dla[2 * k:2 * k + 1] for k in range(4)])
    loc1 = _sel4(zc, *[dla[2 * k + 1:2 * k + 2] for k in range(4)])
    c00 = _sel4(zc, *[dca[4 * k:4 * k + 1] for k in range(4)])
    c10 = _sel4(zc, *[dca[4 * k + 2:4 * k + 3] for k in range(4)])
    c11 = _sel4(zc, *[dca[4 * k + 3:4 * k + 4] for k in range(4)])

    e00 = jnp.exp(c00)
    e11 = jnp.exp(c11)
    ep0 = eps[0:1]
    ep1 = eps[1:2]
    v0 = loc0 + e00 * ep0
    v1 = loc1 + (c10 * ep0 + e11 * ep1)
    np0 = posa[0:1] + v0
    np1 = posa[1:2] + v1

    np_ref[...] = jnp.transpose(jnp.concatenate([np0, np1], axis=0))
    z_ref[...] = zc


@jax.jit
def kernel(position, z, transition, dir_locs, dir_covs, t, displacement,
           W1, b1, W2, b2):
    del z, t
    tr = transition.reshape(P, 16)
    dl = dir_locs.reshape(P, 8)
    dc = dir_covs.reshape(P, 16)

    smem = pl.BlockSpec(memory_space=pltpu.SMEM)
    row = lambda k: pl.BlockSpec((N, k), lambda i: (i, 0))
    new_position, z_row = pl.pallas_call(
        _body,
        grid=(GRID,),
        in_specs=[smem, smem, smem, smem, smem,
                  row(16), row(8), row(16), row(2)],
        out_specs=[row(2), pl.BlockSpec((1, N), lambda i: (0, i))],
        out_shape=[jax.ShapeDtypeStruct((P, 2), jnp.float32),
                   jax.ShapeDtypeStruct((1, P), jnp.int32)],
        compiler_params=pltpu.CompilerParams(
            dimension_semantics=("parallel",)),
    )(displacement, W1, b1, W2, b2, tr, dl, dc, position)

    return (new_position, z_row.reshape(P), transition, dir_locs, dir_covs)
